# baseline (speedup 1.0000x reference)
import jax
import jax.numpy as jnp
from jax import lax
from jax.experimental import pallas as pl
from jax.experimental.pallas import tpu as pltpu
from jax.experimental.pallas import tpu_sc as plsc

NUM_EMBEDDINGS = 100000
EMBEDDING_DIM = 1024

_NUM_CORES = 2
_NUM_SUBCORES = 16
_NUM_WORKERS = _NUM_CORES * _NUM_SUBCORES

_NUM_IDS = 1024 * 50
_IDS_PER_W = _NUM_IDS // _NUM_WORKERS

_CHUNK = 8
_NCHUNK = _IDS_PER_W // _CHUNK
_NBUF = 4


def _gather_body(idx_hbm, table_hbm, out_hbm, idx_v, rows_v, rows_sh, *sems):
    gsem = sems[:_NBUF]
    xsem = sems[_NBUF : 2 * _NBUF]
    osem = sems[2 * _NBUF :]
    sid = lax.axis_index("s")
    wid = sid * _NUM_CORES + lax.axis_index("c")
    base = wid * _IDS_PER_W
    pltpu.sync_copy(idx_hbm.at[pl.ds(base, _IDS_PER_W)], idx_v)

    def _gather(j, buf):
        return pltpu.make_async_copy(
            table_hbm.at[idx_v.at[pl.ds(j * _CHUNK, _CHUNK)]],
            rows_v.at[buf],
            gsem[buf],
        )

    def _xcopy(buf):
        return pltpu.make_async_copy(
            rows_v.at[buf],
            rows_sh.at[sid, buf],
            xsem[buf],
        )

    def _put(j, buf):
        return pltpu.make_async_copy(
            rows_sh.at[sid, buf],
            out_hbm.at[pl.ds(base + j * _CHUNK, _CHUNK)],
            osem[buf],
        )

    for b in range(_NBUF):
        _gather(b, b).start()
    for b in range(_NBUF):
        _gather(b, b).wait()
        _xcopy(b).start()
    for b in range(_NBUF):
        _xcopy(b).wait()
        _put(b, b).start()
        _gather(b + _NBUF, b).start()

    @pl.loop(_NBUF, _NCHUNK - _NBUF, step=_NBUF)
    def _group(m):
        for b in range(_NBUF):
            _gather(m + b, b).wait()
            _put(m - _NBUF + b, b).wait()
            _xcopy(b).start()
        for b in range(_NBUF):
            _xcopy(b).wait()
            _put(m + b, b).start()
            _gather(m + b + _NBUF, b).start()

    for b in range(_NBUF):
        _gather(_NCHUNK - _NBUF + b, b).wait()
        _put(_NCHUNK - 2 * _NBUF + b, b).wait()
        _xcopy(b).start()
    for b in range(_NBUF):
        _xcopy(b).wait()
        _put(_NCHUNK - _NBUF + b, b).start()
    for b in range(_NBUF):
        _put(_NCHUNK - _NBUF + b, b).wait()


@jax.jit
def _encode(input_ids, table):
    mesh = plsc.VectorSubcoreMesh(core_axis_name="c", subcore_axis_name="s")
    flat = pl.kernel(
        _gather_body,
        out_type=jax.ShapeDtypeStruct((_NUM_IDS, EMBEDDING_DIM), jnp.float32),
        mesh=mesh,
        scratch_types=[
            pltpu.VMEM((_IDS_PER_W,), jnp.int32),
            pltpu.VMEM((_NBUF, _CHUNK, EMBEDDING_DIM), jnp.float32),
            pltpu.VMEM_SHARED(
                (_NUM_SUBCORES, _NBUF, _CHUNK, EMBEDDING_DIM), jnp.float32
            ),
        ]
        + [pltpu.SemaphoreType.DMA] * (3 * _NBUF),
    )(input_ids.reshape(-1), table)
    return flat.reshape(input_ids.shape[0], input_ids.shape[1], EMBEDDING_DIM)


def kernel(input_ids, table):
    return _encode(input_ids, table)
